# Initial kernel scaffold; baseline (speedup 1.0000x reference)
import jax
import jax.numpy as jnp
from jax.experimental import pallas as pl


def kernel(x, G2_edge_attr, G1_edge_attr_matrix, G3_edge_index, G3_edge_attr, W1, b1, W2, b2):
    raise NotImplementedError("write your pallas kernel here")


# speedup vs baseline: 13.7196x; 13.7196x over previous
import functools

import jax
import jax.numpy as jnp
from jax import lax
from jax.experimental import pallas as pl
from jax.experimental.pallas import tpu as pltpu
from jax.experimental.pallas import tpu_sc as plsc

N = 10000
E = 320000
D_IN = 128
D_HID = 64
D_OUT = 16

NC = 2
NS = 16
NW = NC * NS
CHUNK = 512
EPT = 10240
EPAD = NW * EPT
NCHUNK = EPT // CHUNK
NACC = 10240
RPT = NACC // NS

_mesh = plsc.VectorSubcoreMesh(core_axis_name="c", subcore_axis_name="s")


def _sc_deg_body(dst2d, ew, zeros, out, dstv, ewv, sem, deg_sh):
    c = lax.axis_index("c")
    s = lax.axis_index("s")
    w = c * NS + s
    pltpu.sync_copy(zeros.at[pl.ds(s * RPT, RPT)], deg_sh.at[pl.ds(s * RPT, RPT)])
    plsc.subcore_barrier()
    for k in range(NCHUNK):
        rb = w * (EPT // 128) + k * (CHUNK // 128)
        base = w * EPT + k * CHUNK
        pltpu.sync_copy(dst2d.at[pl.ds(rb, CHUNK // 128), :], dstv)
        pltpu.sync_copy(ew.at[pl.ds(base, CHUNK)], ewv)
        for j in range(CHUNK // 128):
            pltpu.sync_copy(ewv.at[pl.ds(j * 128, 128)],
                            deg_sh.at[dstv.at[j]], add=True)
    plsc.subcore_barrier()
    pltpu.sync_copy(deg_sh.at[pl.ds(s * RPT, RPT)], out.at[c, pl.ds(s * RPT, RPT)])


def _sc_agg_body(d, src2d, dst2d, ew, g, zeros, out, srcv, dstv, ewv, rows, sem,
                 acc_sh):
    c = lax.axis_index("c")
    s = lax.axis_index("s")
    w = c * NS + s
    pltpu.sync_copy(zeros.at[pl.ds(s * RPT, RPT), :],
                    acc_sh.at[pl.ds(s * RPT, RPT), :])
    plsc.subcore_barrier()
    for k in range(NCHUNK):
        rb = w * (EPT // 128) + k * (CHUNK // 128)
        base = w * EPT + k * CHUNK
        pltpu.sync_copy(src2d.at[pl.ds(rb, CHUNK // 128), :], srcv)
        pltpu.sync_copy(dst2d.at[pl.ds(rb, CHUNK // 128), :], dstv)
        pltpu.sync_copy(ew.at[pl.ds(base, CHUNK)], ewv.at[pl.ds(0, CHUNK)])
        for j in range(CHUNK // 128):
            pltpu.async_copy(g.at[srcv.at[j]],
                             rows.at[pl.ds(j * 128, 128), :], sem).wait()

        def scale(i, _):
            ws = ewv[pl.ds(i, 16)][0]
            for jj in range(d // 16):
                rows[i, pl.ds(jj * 16, 16)] = rows[i, pl.ds(jj * 16, 16)] * ws
            return 0

        lax.fori_loop(0, CHUNK, scale, 0)
        for j in range(CHUNK // 128):
            pltpu.sync_copy(rows.at[pl.ds(j * 128, 128), :],
                            acc_sh.at[dstv.at[j]], add=True)
    plsc.subcore_barrier()
    pltpu.sync_copy(acc_sh.at[pl.ds(s * RPT, RPT), :],
                    out.at[c, pl.ds(s * RPT, RPT), :])


def _sc_deg(dst2d, ew, zeros):
    return pl.kernel(
        _sc_deg_body,
        out_type=jax.ShapeDtypeStruct((NC, NACC), jnp.float32),
        mesh=_mesh,
        scratch_types=[
            pltpu.VMEM((CHUNK // 128, 128), jnp.int32),
            pltpu.VMEM((CHUNK,), jnp.float32),
            pltpu.SemaphoreType.DMA,
            pltpu.VMEM_SHARED((NACC,), jnp.float32),
        ],
        name="gcn_sc_deg",
    )(dst2d, ew, zeros)


def _sc_agg(d, src2d, dst2d, ew, g, zeros):
    return pl.kernel(
        functools.partial(_sc_agg_body, d),
        out_type=jax.ShapeDtypeStruct((NC, NACC, d), jnp.float32),
        mesh=_mesh,
        scratch_types=[
            pltpu.VMEM((CHUNK // 128, 128), jnp.int32),
            pltpu.VMEM((CHUNK // 128, 128), jnp.int32),
            pltpu.VMEM((CHUNK + 16,), jnp.float32),
            pltpu.VMEM((CHUNK, d), jnp.float32),
            pltpu.SemaphoreType.DMA,
            pltpu.VMEM_SHARED((NACC, d), jnp.float32),
        ],
        compiler_params=pltpu.CompilerParams(use_tc_tiling_on_sc=False),
        name=f"gcn_sc_agg{d}",
    )(src2d, dst2d, ew, g, zeros)


def _tc_pre_body(deg_ref, x_ref, w1_ref, g1_ref, dinv_ref):
    deg = deg_ref[0] + deg_ref[1]
    dinv = lax.rsqrt(deg + 1.0)
    dinv_ref[...] = dinv
    h = jnp.dot(x_ref[...], w1_ref[...], preferred_element_type=jnp.float32)
    g1_ref[...] = h * dinv[:N]


def _tc_mid_body(s1_ref, g1_ref, dinv_ref, b1_ref, w2_ref, g2_ref):
    s1 = s1_ref[0, :N] + s1_ref[1, :N]
    dinv = dinv_ref[:N]
    out1 = jnp.maximum(dinv * (s1 + g1_ref[...]) + b1_ref[...], 0.0)
    h2 = jnp.dot(out1, w2_ref[...], preferred_element_type=jnp.float32)
    g2_ref[...] = h2 * dinv


def _tc_out_body(s2_ref, g2_ref, dinv_ref, b2_ref, o_ref):
    s2 = s2_ref[0, :N] + s2_ref[1, :N]
    z = dinv_ref[:N] * (s2 + g2_ref[...]) + b2_ref[...]
    m = jnp.max(z, axis=1, keepdims=True)
    lse = m + jnp.log(jnp.sum(jnp.exp(z - m), axis=1, keepdims=True))
    o_ref[...] = z - lse


def kernel(x, G2_edge_attr, G1_edge_attr_matrix, G3_edge_index, G3_edge_attr,
           W1, b1, W2, b2):
    src = G3_edge_index[0]
    dst = G3_edge_index[1]
    ew = G3_edge_attr
    pad = EPAD - E
    zi = jnp.zeros((pad,), jnp.int32)
    src2d = jnp.concatenate([src, zi]).reshape(EPAD // 128, 128)
    dst2d = jnp.concatenate([dst, zi]).reshape(EPAD // 128, 128)
    ewp = jnp.concatenate([ew, jnp.zeros((pad,), jnp.float32)])

    deg_part = _sc_deg(dst2d, ewp, jnp.zeros((NACC,), jnp.float32))
    deg_col = deg_part.reshape(NC, NACC, 1)

    g1, dinv = pl.pallas_call(
        _tc_pre_body,
        out_shape=(jax.ShapeDtypeStruct((N, D_HID), jnp.float32),
                   jax.ShapeDtypeStruct((NACC, 1), jnp.float32)),
    )(deg_col, x, W1)

    s1 = _sc_agg(D_HID, src2d, dst2d, ewp, g1,
                 jnp.zeros((NACC, D_HID), jnp.float32))

    g2 = pl.pallas_call(
        _tc_mid_body,
        out_shape=jax.ShapeDtypeStruct((N, D_OUT), jnp.float32),
    )(s1, g1, dinv, b1.reshape(1, D_HID), W2)

    s2 = _sc_agg(D_OUT, src2d, dst2d, ewp, g2,
                 jnp.zeros((NACC, D_OUT), jnp.float32))

    out = pl.pallas_call(
        _tc_out_body,
        out_shape=jax.ShapeDtypeStruct((N, D_OUT), jnp.float32),
    )(s2, g2, dinv, b2.reshape(1, D_OUT))
    return out


# speedup vs baseline: 20.8328x; 1.5185x over previous
import functools

import jax
import jax.numpy as jnp
from jax import lax
from jax.experimental import pallas as pl
from jax.experimental.pallas import tpu as pltpu
from jax.experimental.pallas import tpu_sc as plsc

N = 10000
E = 320000
D_IN = 128
D_HID = 64
D_OUT = 16

NC = 2
NS = 16
NW = NC * NS
CHUNK = 512
EPT = 10240
EPAD = NW * EPT
NCHUNK = EPT // CHUNK
NACC = 10240
RPT = NACC // NS

_mesh = plsc.VectorSubcoreMesh(core_axis_name="c", subcore_axis_name="s")


def _sc_deg_body(dst2d, ew, zeros, out, dstv, ewv, sem, deg_sh):
    c = lax.axis_index("c")
    s = lax.axis_index("s")
    w = c * NS + s
    pltpu.sync_copy(zeros.at[pl.ds(s * RPT, RPT)], deg_sh.at[pl.ds(s * RPT, RPT)])
    plsc.subcore_barrier()
    for k in range(NCHUNK):
        rb = w * (EPT // 128) + k * (CHUNK // 128)
        base = w * EPT + k * CHUNK
        pltpu.sync_copy(dst2d.at[pl.ds(rb, CHUNK // 128), :], dstv)
        pltpu.sync_copy(ew.at[pl.ds(base, CHUNK)], ewv)
        for j in range(CHUNK // 128):
            pltpu.sync_copy(ewv.at[pl.ds(j * 128, 128)],
                            deg_sh.at[dstv.at[j]], add=True)
    plsc.subcore_barrier()
    pltpu.sync_copy(deg_sh.at[pl.ds(s * RPT, RPT)], out.at[c, pl.ds(s * RPT, RPT)])


def _sc_agg_body(d, src2d, dst2d, ew, g, zeros, out, srcv, dstv, ewv, rows,
                 sem_i, sem_g, sem_s, acc_sh):
    c = lax.axis_index("c")
    s = lax.axis_index("s")
    w = c * NS + s
    pltpu.sync_copy(zeros.at[pl.ds(s * RPT, RPT), :],
                    acc_sh.at[pl.ds(s * RPT, RPT), :])
    plsc.subcore_barrier()

    def start_idx(k, b):
        rb = w * (EPT // 128) + k * (CHUNK // 128)
        base = w * EPT + k * CHUNK
        return (
            pltpu.async_copy(src2d.at[pl.ds(rb, CHUNK // 128), :],
                             srcv.at[b], sem_i),
            pltpu.async_copy(dst2d.at[pl.ds(rb, CHUNK // 128), :],
                             dstv.at[b], sem_i),
            pltpu.async_copy(ew.at[pl.ds(base, CHUNK)],
                             ewv.at[b, pl.ds(0, CHUNK)], sem_i),
        )

    def start_gather(k, rb):
        i3 = k % 3
        return [
            pltpu.async_copy(g.at[srcv.at[i3, j]],
                             rows.at[rb, pl.ds(j * 128, 128), :], sem_g)
            for j in range(CHUNK // 128)
        ]

    def start_scatter(k):
        i3 = k % 3
        rb = k % 2
        return [
            pltpu.async_copy(rows.at[rb, pl.ds(j * 128, 128), :],
                             acc_sh.at[dstv.at[i3, j]], sem_s, add=True)
            for j in range(CHUNK // 128)
        ]

    def scale_chunk(k):
        i3 = k % 3
        rb = k % 2

        def scale(i, _):
            ws = ewv[i3, pl.ds(i, 16)][0]
            for jj in range(d // 16):
                rows[rb, i, pl.ds(jj * 16, 16)] = (
                    rows[rb, i, pl.ds(jj * 16, 16)] * ws)
            return 0
        lax.fori_loop(0, CHUNK, scale, 0, unroll=8)

    idx_d = start_idx(0, 0)
    for cp in idx_d:
        cp.wait()
    g_d = start_gather(0, 0)
    sc_d = None
    for k in range(NCHUNK):
        b = k % 2
        if k + 1 < NCHUNK:
            idx_d = start_idx(k + 1, (k + 1) % 3)
        for cp in g_d:
            cp.wait()
        if k + 1 < NCHUNK:
            for cp in idx_d:
                cp.wait()
            if sc_d is not None:
                for cp in sc_d:
                    cp.wait()
                sc_d = None
            g_d = start_gather(k + 1, (k + 1) % 2)
        scale_chunk(k)
        if sc_d is not None:
            for cp in sc_d:
                cp.wait()
        sc_d = start_scatter(k)
    for cp in sc_d:
        cp.wait()
    plsc.subcore_barrier()
    pltpu.sync_copy(acc_sh.at[pl.ds(s * RPT, RPT), :],
                    out.at[c, pl.ds(s * RPT, RPT), :])


def _sc_deg(dst2d, ew, zeros):
    return pl.kernel(
        _sc_deg_body,
        out_type=jax.ShapeDtypeStruct((NC, NACC), jnp.float32),
        mesh=_mesh,
        scratch_types=[
            pltpu.VMEM((CHUNK // 128, 128), jnp.int32),
            pltpu.VMEM((CHUNK,), jnp.float32),
            pltpu.SemaphoreType.DMA,
            pltpu.VMEM_SHARED((NACC,), jnp.float32),
        ],
        name="gcn_sc_deg",
    )(dst2d, ew, zeros)


def _sc_agg(d, src2d, dst2d, ew, g, zeros):
    return pl.kernel(
        functools.partial(_sc_agg_body, d),
        out_type=jax.ShapeDtypeStruct((NC, NACC, d), jnp.float32),
        mesh=_mesh,
        scratch_types=[
            pltpu.VMEM((3, CHUNK // 128, 128), jnp.int32),
            pltpu.VMEM((3, CHUNK // 128, 128), jnp.int32),
            pltpu.VMEM((3, CHUNK + 16), jnp.float32),
            pltpu.VMEM((2, CHUNK, d), jnp.float32),
            pltpu.SemaphoreType.DMA,
            pltpu.SemaphoreType.DMA,
            pltpu.SemaphoreType.DMA,
            pltpu.VMEM_SHARED((NACC, d), jnp.float32),
        ],
        compiler_params=pltpu.CompilerParams(use_tc_tiling_on_sc=False),
        name=f"gcn_sc_agg{d}",
    )(src2d, dst2d, ew, g, zeros)


def _tc_pre_body(deg_ref, x_ref, w1_ref, g1_ref, dinv_ref):
    deg = deg_ref[0] + deg_ref[1]
    dinv = lax.rsqrt(deg + 1.0)
    dinv_ref[...] = dinv
    h = jnp.dot(x_ref[...], w1_ref[...], preferred_element_type=jnp.float32)
    g1_ref[...] = h * dinv[:N]


def _tc_mid_body(s1_ref, g1_ref, dinv_ref, b1_ref, w2_ref, g2_ref):
    s1 = s1_ref[0, :N] + s1_ref[1, :N]
    dinv = dinv_ref[:N]
    out1 = jnp.maximum(dinv * (s1 + g1_ref[...]) + b1_ref[...], 0.0)
    h2 = jnp.dot(out1, w2_ref[...], preferred_element_type=jnp.float32)
    g2_ref[...] = h2 * dinv


def _tc_out_body(s2_ref, g2_ref, dinv_ref, b2_ref, o_ref):
    s2 = s2_ref[0, :N] + s2_ref[1, :N]
    z = dinv_ref[:N] * (s2 + g2_ref[...]) + b2_ref[...]
    m = jnp.max(z, axis=1, keepdims=True)
    lse = m + jnp.log(jnp.sum(jnp.exp(z - m), axis=1, keepdims=True))
    o_ref[...] = z - lse


def kernel(x, G2_edge_attr, G1_edge_attr_matrix, G3_edge_index, G3_edge_attr,
           W1, b1, W2, b2):
    src = G3_edge_index[0]
    dst = G3_edge_index[1]
    ew = G3_edge_attr
    pad = EPAD - E
    zi = jnp.zeros((pad,), jnp.int32)
    src2d = jnp.concatenate([src, zi]).reshape(EPAD // 128, 128)
    dst2d = jnp.concatenate([dst, zi]).reshape(EPAD // 128, 128)
    ewp = jnp.concatenate([ew, jnp.zeros((pad,), jnp.float32)])

    deg_part = _sc_deg(dst2d, ewp, jnp.zeros((NACC,), jnp.float32))
    deg_col = deg_part.reshape(NC, NACC, 1)

    g1, dinv = pl.pallas_call(
        _tc_pre_body,
        out_shape=(jax.ShapeDtypeStruct((N, D_HID), jnp.float32),
                   jax.ShapeDtypeStruct((NACC, 1), jnp.float32)),
    )(deg_col, x, W1)

    s1 = _sc_agg(D_HID, src2d, dst2d, ewp, g1,
                 jnp.zeros((NACC, D_HID), jnp.float32))

    g2 = pl.pallas_call(
        _tc_mid_body,
        out_shape=jax.ShapeDtypeStruct((N, D_OUT), jnp.float32),
    )(s1, g1, dinv, b1.reshape(1, D_HID), W2)

    s2 = _sc_agg(D_OUT, src2d, dst2d, ewp, g2,
                 jnp.zeros((NACC, D_OUT), jnp.float32))

    out = pl.pallas_call(
        _tc_out_body,
        out_shape=jax.ShapeDtypeStruct((N, D_OUT), jnp.float32),
    )(s2, g2, dinv, b2.reshape(1, D_OUT))
    return out


# speedup vs baseline: 24.8327x; 1.1920x over previous
import functools

import jax
import jax.numpy as jnp
from jax import lax
from jax.experimental import pallas as pl
from jax.experimental.pallas import tpu as pltpu
from jax.experimental.pallas import tpu_sc as plsc

N = 10000
E = 320000
D_IN = 128
D_HID = 64
D_OUT = 16

NC = 2
NS = 16
NW = NC * NS
CHUNK = 512
EPT = 10240
EPAD = NW * EPT
NCHUNK = EPT // CHUNK
NACC = 10240
RPT = NACC // NS

_mesh = plsc.VectorSubcoreMesh(core_axis_name="c", subcore_axis_name="s")


def _sc_deg_body(dst2d, ew, zeros, out, dstv, ewv, sem, deg_sh):
    c = lax.axis_index("c")
    s = lax.axis_index("s")
    w = c * NS + s
    pltpu.sync_copy(zeros.at[pl.ds(s * RPT, RPT)], deg_sh.at[pl.ds(s * RPT, RPT)])
    plsc.subcore_barrier()
    for k in range(NCHUNK):
        rb = w * (EPT // 128) + k * (CHUNK // 128)
        base = w * EPT + k * CHUNK
        pltpu.sync_copy(dst2d.at[pl.ds(rb, CHUNK // 128), :], dstv)
        pltpu.sync_copy(ew.at[pl.ds(base, CHUNK)], ewv)
        for j in range(CHUNK // 128):
            pltpu.sync_copy(ewv.at[pl.ds(j * 128, 128)],
                            deg_sh.at[dstv.at[j]], add=True)
    plsc.subcore_barrier()
    pltpu.sync_copy(deg_sh.at[pl.ds(s * RPT, RPT)], out.at[c, pl.ds(s * RPT, RPT)])


def _sc_agg_body(d, packed, chunk, *refs):
    if packed:
        (src2d, dst2d, ew, g, zeros, out, srcv, dstv, ewv, rows, rows_b,
         sem_i, sem_g, sem_s, acc_sh) = refs
    else:
        (src2d, dst2d, ew, g, zeros, out, srcv, dstv, ewv, rows,
         sem_i, sem_g, sem_s, acc_sh) = refs
        rows_b = rows
    c = lax.axis_index("c")
    s = lax.axis_index("s")
    w = c * NS + s
    pltpu.sync_copy(zeros.at[pl.ds(s * RPT, RPT), :],
                    acc_sh.at[pl.ds(s * RPT, RPT), :])
    plsc.subcore_barrier()
    nchunk = EPT // chunk

    def start_idx(k, b):
        rb = w * (EPT // 128) + k * (chunk // 128)
        base = w * EPT + k * chunk
        return (
            pltpu.async_copy(src2d.at[pl.ds(rb, chunk // 128), :],
                             srcv.at[b], sem_i),
            pltpu.async_copy(dst2d.at[pl.ds(rb, chunk // 128), :],
                             dstv.at[b], sem_i),
            pltpu.async_copy(ew.at[pl.ds(base, chunk)],
                             ewv.at[b, pl.ds(0, chunk)], sem_i),
        )

    def start_gather(k, rb):
        i3 = k % 3
        return [
            pltpu.async_copy(g.at[srcv.at[i3, j]],
                             rows_b.at[rb, pl.ds(j * 128, 128), :], sem_g)
            for j in range(chunk // 128)
        ]

    def start_scatter(k):
        i3 = k % 3
        rb = k % 2
        return [
            pltpu.async_copy(rows.at[rb, pl.ds(j * 128, 128), :],
                             acc_sh.at[dstv.at[i3, j]], sem_s, add=True)
            for j in range(chunk // 128)
        ]

    def scale_chunk(k):
        i3 = k % 3
        rb = k % 2

        def scale(i, _):
            ws = ewv[i3, pl.ds(i, 16)][0]
            if packed:
                for jj in range(d // 32):
                    wv = rows_b[rb, i, pl.ds(jj * 16, 16)]
                    e0 = lax.bitcast_convert_type(wv << 16, jnp.float32)
                    e1 = lax.bitcast_convert_type(wv & jnp.int32(-65536),
                                                  jnp.float32)
                    rows[rb, i, pl.ds(jj * 16, 16)] = e0 * ws
                    rows[rb, i, pl.ds(d // 2 + jj * 16, 16)] = e1 * ws
            else:
                for jj in range(d // 16):
                    rows[rb, i, pl.ds(jj * 16, 16)] = (
                        rows[rb, i, pl.ds(jj * 16, 16)] * ws)
            return 0
        lax.fori_loop(0, chunk, scale, 0, unroll=4)

    idx_d = start_idx(0, 0)
    for cp in idx_d:
        cp.wait()
    g_d = start_gather(0, 0)
    sc_d = None
    for k in range(nchunk):
        b = k % 2
        if k + 1 < nchunk:
            idx_d = start_idx(k + 1, (k + 1) % 3)
        for cp in g_d:
            cp.wait()
        if k + 1 < nchunk:
            for cp in idx_d:
                cp.wait()
            if sc_d is not None:
                for cp in sc_d:
                    cp.wait()
                sc_d = None
            g_d = start_gather(k + 1, (k + 1) % 2)
        scale_chunk(k)
        if sc_d is not None:
            for cp in sc_d:
                cp.wait()
        sc_d = start_scatter(k)
    for cp in sc_d:
        cp.wait()
    plsc.subcore_barrier()
    pltpu.sync_copy(acc_sh.at[pl.ds(s * RPT, RPT), :],
                    out.at[c, pl.ds(s * RPT, RPT), :])


def _sc_deg(dst2d, ew, zeros):
    return pl.kernel(
        _sc_deg_body,
        out_type=jax.ShapeDtypeStruct((NC, NACC), jnp.float32),
        mesh=_mesh,
        scratch_types=[
            pltpu.VMEM((CHUNK // 128, 128), jnp.int32),
            pltpu.VMEM((CHUNK,), jnp.float32),
            pltpu.SemaphoreType.DMA,
            pltpu.VMEM_SHARED((NACC,), jnp.float32),
        ],
        name="gcn_sc_deg",
    )(dst2d, ew, zeros)


def _sc_agg(d, packed, chunk, src2d, dst2d, ew, g, zeros):
    scratch = [
        pltpu.VMEM((3, chunk // 128, 128), jnp.int32),
        pltpu.VMEM((3, chunk // 128, 128), jnp.int32),
        pltpu.VMEM((3, chunk + 16), jnp.float32),
        pltpu.VMEM((2, chunk, d), jnp.float32),
    ]
    if packed:
        scratch.append(pltpu.VMEM((2, chunk, d // 2), jnp.int32))
    scratch += [
        pltpu.SemaphoreType.DMA,
        pltpu.SemaphoreType.DMA,
        pltpu.SemaphoreType.DMA,
        pltpu.VMEM_SHARED((NACC, d), jnp.float32),
    ]
    return pl.kernel(
        functools.partial(_sc_agg_body, d, packed, chunk),
        out_type=jax.ShapeDtypeStruct((NC, NACC, d), jnp.float32),
        mesh=_mesh,
        scratch_types=scratch,
        compiler_params=pltpu.CompilerParams(use_tc_tiling_on_sc=False),
        name=f"gcn_sc_agg{d}",
    )(src2d, dst2d, ew, g, zeros)


def _tc_pre_body(deg_ref, x_ref, w1_ref, g1_ref, g1b_ref, dinv_ref):
    deg = deg_ref[0] + deg_ref[1]
    dinv = lax.rsqrt(deg + 1.0)
    dinv_ref[...] = dinv
    h = jnp.dot(x_ref[...], w1_ref[...], preferred_element_type=jnp.float32)
    g1 = h * dinv[:N]
    g1_ref[...] = g1
    lo = lax.convert_element_type(
        lax.bitcast_convert_type(g1[:, :D_HID // 2].astype(jnp.bfloat16),
                                 jnp.uint16), jnp.int32)
    hi = lax.convert_element_type(
        lax.bitcast_convert_type(g1[:, D_HID // 2:].astype(jnp.bfloat16),
                                 jnp.uint16), jnp.int32)
    g1b_ref[...] = lo | (hi << 16)


def _tc_mid_body(s1_ref, g1_ref, dinv_ref, b1_ref, w2_ref, g2_ref):
    s1 = s1_ref[0, :N] + s1_ref[1, :N]
    dinv = dinv_ref[:N]
    out1 = jnp.maximum(dinv * (s1 + g1_ref[...]) + b1_ref[...], 0.0)
    h2 = jnp.dot(out1, w2_ref[...], preferred_element_type=jnp.float32)
    g2_ref[...] = h2 * dinv


def _tc_out_body(s2_ref, g2_ref, dinv_ref, b2_ref, o_ref):
    s2 = s2_ref[0, :N] + s2_ref[1, :N]
    z = dinv_ref[:N] * (s2 + g2_ref[...]) + b2_ref[...]
    m = jnp.max(z, axis=1, keepdims=True)
    lse = m + jnp.log(jnp.sum(jnp.exp(z - m), axis=1, keepdims=True))
    o_ref[...] = z - lse


def kernel(x, G2_edge_attr, G1_edge_attr_matrix, G3_edge_index, G3_edge_attr,
           W1, b1, W2, b2):
    src = G3_edge_index[0]
    dst = G3_edge_index[1]
    ew = G3_edge_attr
    pad = EPAD - E
    zi = jnp.zeros((pad,), jnp.int32)
    src2d = jnp.concatenate([src, zi]).reshape(EPAD // 128, 128)
    dst2d = jnp.concatenate([dst, zi]).reshape(EPAD // 128, 128)
    ewp = jnp.concatenate([ew, jnp.zeros((pad,), jnp.float32)])

    deg_part = _sc_deg(dst2d, ewp, jnp.zeros((NACC,), jnp.float32))
    deg_col = deg_part.reshape(NC, NACC, 1)

    g1, g1b, dinv = pl.pallas_call(
        _tc_pre_body,
        out_shape=(jax.ShapeDtypeStruct((N, D_HID), jnp.float32),
                   jax.ShapeDtypeStruct((N, D_HID // 2), jnp.int32),
                   jax.ShapeDtypeStruct((NACC, 1), jnp.float32)),
    )(deg_col, x, W1)

    s1 = _sc_agg(D_HID, True, 256, src2d, dst2d, ewp, g1b,
                 jnp.zeros((NACC, D_HID), jnp.float32))

    g2 = pl.pallas_call(
        _tc_mid_body,
        out_shape=jax.ShapeDtypeStruct((N, D_OUT), jnp.float32),
    )(s1, g1, dinv, b1.reshape(1, D_HID), W2)

    s2 = _sc_agg(D_OUT, False, 512, src2d, dst2d, ewp, g2,
                 jnp.zeros((NACC, D_OUT), jnp.float32))

    out = pl.pallas_call(
        _tc_out_body,
        out_shape=jax.ShapeDtypeStruct((N, D_OUT), jnp.float32),
    )(s2, g2, dinv, b2.reshape(1, D_OUT))
    return out


# speedup vs baseline: 25.4140x; 1.0234x over previous
import functools

import jax
import jax.numpy as jnp
from jax import lax
from jax.experimental import pallas as pl
from jax.experimental.pallas import tpu as pltpu
from jax.experimental.pallas import tpu_sc as plsc

N = 10000
E = 320000
D_IN = 128
D_HID = 64
D_OUT = 16

NC = 2
NS = 16
NW = NC * NS
CHUNK = 512
EPT = 10240
EPAD = NW * EPT
NCHUNK = EPT // CHUNK
NACC = 10240
RPT = NACC // NS

_mesh = plsc.VectorSubcoreMesh(core_axis_name="c", subcore_axis_name="s")


def _sc_deg_body(dst2d, ew, zeros, out, dstv, ewv, sem, deg_sh):
    c = lax.axis_index("c")
    s = lax.axis_index("s")
    w = c * NS + s
    pltpu.sync_copy(zeros.at[pl.ds(s * RPT, RPT)], deg_sh.at[pl.ds(s * RPT, RPT)])
    plsc.subcore_barrier()
    for k in range(NCHUNK):
        rb = w * (EPT // 128) + k * (CHUNK // 128)
        base = w * EPT + k * CHUNK
        pltpu.sync_copy(dst2d.at[pl.ds(rb, CHUNK // 128), :], dstv)
        pltpu.sync_copy(ew.at[pl.ds(base, CHUNK)], ewv)
        for j in range(CHUNK // 128):
            pltpu.sync_copy(ewv.at[pl.ds(j * 128, 128)],
                            deg_sh.at[dstv.at[j]], add=True)
    plsc.subcore_barrier()
    pltpu.sync_copy(deg_sh.at[pl.ds(s * RPT, RPT)], out.at[c, pl.ds(s * RPT, RPT)])


def _sc_agg_body(d, packed, chunk, *refs):
    if packed:
        (src2d, dst2d, ew, g, zeros, out, srcv, dstv, ewv, rows, rows_b,
         sem_i, sem_g, sem_s, acc_sh) = refs
    else:
        (src2d, dst2d, ew, g, zeros, out, srcv, dstv, ewv, rows,
         sem_i, sem_g, sem_s, acc_sh) = refs
        rows_b = rows
    c = lax.axis_index("c")
    s = lax.axis_index("s")
    w = c * NS + s
    pltpu.sync_copy(zeros.at[pl.ds(s * RPT, RPT), :],
                    acc_sh.at[pl.ds(s * RPT, RPT), :])
    plsc.subcore_barrier()
    nchunk = EPT // chunk
    nj = chunk // 128
    G = 4
    outer = nchunk // G

    def start_idx(k, b):
        rb = w * (EPT // 128) + k * nj
        base = w * EPT + k * chunk
        pltpu.async_copy(src2d.at[pl.ds(rb, nj), :], srcv.at[b], sem_i)
        pltpu.async_copy(dst2d.at[pl.ds(rb, nj), :], dstv.at[b], sem_i)
        pltpu.async_copy(ew.at[pl.ds(base, chunk)],
                         ewv.at[b, pl.ds(0, chunk)], sem_i)

    def wait_idx(b):
        pltpu.make_async_copy(src2d.at[pl.ds(0, nj), :], srcv.at[b],
                              sem_i).wait()
        pltpu.make_async_copy(dst2d.at[pl.ds(0, nj), :], dstv.at[b],
                              sem_i).wait()
        pltpu.make_async_copy(ew.at[pl.ds(0, chunk)],
                              ewv.at[b, pl.ds(0, chunk)], sem_i).wait()

    def start_gather(b, rb):
        for j in range(nj):
            pltpu.async_copy(g.at[srcv.at[b, j]],
                             rows_b.at[rb, pl.ds(j * 128, 128), :], sem_g)

    def wait_gather(b, rb):
        for j in range(nj):
            pltpu.make_async_copy(g.at[srcv.at[b, j]],
                                  rows_b.at[rb, pl.ds(j * 128, 128), :],
                                  sem_g).wait()

    def start_scatter(b, rb):
        for j in range(nj):
            pltpu.async_copy(rows.at[rb, pl.ds(j * 128, 128), :],
                             acc_sh.at[dstv.at[b, j]], sem_s, add=True)

    def wait_scatter(b, rb):
        for j in range(nj):
            pltpu.make_async_copy(rows.at[rb, pl.ds(j * 128, 128), :],
                                  acc_sh.at[dstv.at[b, j]], sem_s).wait()

    def scale_chunk(b, rb):
        def scale(i, _):
            ws = ewv[b, pl.ds(i, 16)][0]
            if packed:
                for jj in range(d // 32):
                    wv = rows_b[rb, i, pl.ds(jj * 16, 16)]
                    e0 = lax.bitcast_convert_type(wv << 16, jnp.float32)
                    e1 = lax.bitcast_convert_type(wv & jnp.int32(-65536),
                                                  jnp.float32)
                    rows[rb, i, pl.ds(jj * 16, 16)] = e0 * ws
                    rows[rb, i, pl.ds(d // 2 + jj * 16, 16)] = e1 * ws
            else:
                for jj in range(d // 16):
                    rows[rb, i, pl.ds(jj * 16, 16)] = (
                        rows[rb, i, pl.ds(jj * 16, 16)] * ws)
            return 0
        lax.fori_loop(0, chunk, scale, 0, unroll=8)

    start_idx(0, 0)
    wait_idx(0)
    start_gather(0, 0)

    def outer_body(kk, _):
        for gg in range(G):
            k = kk * G + gg
            start_idx(k + 1, (gg + 1) % G)
            wait_gather(gg, gg % 2)
            wait_idx((gg + 1) % G)

            @pl.when(k >= 1)
            def _():
                wait_scatter((gg + 1) % G, (gg + 1) % 2)
            start_gather((gg + 1) % G, (gg + 1) % 2)
            scale_chunk(gg, gg % 2)
            start_scatter(gg, gg % 2)
        return 0

    lax.fori_loop(0, outer - 1, outer_body, 0)

    for gg in range(G):
        k = (outer - 1) * G + gg
        if gg + 1 < G:
            start_idx(k + 1, gg + 1)
        wait_gather(gg, gg % 2)
        if gg + 1 < G:
            wait_idx(gg + 1)
            wait_scatter(gg + 1, (gg + 1) % 2)
            start_gather(gg + 1, (gg + 1) % 2)
        scale_chunk(gg, gg % 2)
        if gg + 1 == G:
            wait_scatter((gg + 1) % G, (gg + 1) % 2)
        start_scatter(gg, gg % 2)
    wait_scatter(G - 1, (G - 1) % 2)
    plsc.subcore_barrier()
    pltpu.sync_copy(acc_sh.at[pl.ds(s * RPT, RPT), :],
                    out.at[c, pl.ds(s * RPT, RPT), :])


def _sc_deg(dst2d, ew, zeros):
    return pl.kernel(
        _sc_deg_body,
        out_type=jax.ShapeDtypeStruct((NC, NACC), jnp.float32),
        mesh=_mesh,
        scratch_types=[
            pltpu.VMEM((CHUNK // 128, 128), jnp.int32),
            pltpu.VMEM((CHUNK,), jnp.float32),
            pltpu.SemaphoreType.DMA,
            pltpu.VMEM_SHARED((NACC,), jnp.float32),
        ],
        name="gcn_sc_deg",
    )(dst2d, ew, zeros)


def _sc_agg(d, packed, chunk, src2d, dst2d, ew, g, zeros):
    scratch = [
        pltpu.VMEM((4, chunk // 128, 128), jnp.int32),
        pltpu.VMEM((4, chunk // 128, 128), jnp.int32),
        pltpu.VMEM((4, chunk + 16), jnp.float32),
        pltpu.VMEM((2, chunk, d), jnp.float32),
    ]
    if packed:
        scratch.append(pltpu.VMEM((2, chunk, d // 2), jnp.int32))
    scratch += [
        pltpu.SemaphoreType.DMA,
        pltpu.SemaphoreType.DMA,
        pltpu.SemaphoreType.DMA,
        pltpu.VMEM_SHARED((NACC, d), jnp.float32),
    ]
    return pl.kernel(
        functools.partial(_sc_agg_body, d, packed, chunk),
        out_type=jax.ShapeDtypeStruct((NC, NACC, d), jnp.float32),
        mesh=_mesh,
        scratch_types=scratch,
        compiler_params=pltpu.CompilerParams(use_tc_tiling_on_sc=False),
        name=f"gcn_sc_agg{d}",
    )(src2d, dst2d, ew, g, zeros)


def _tc_pre_body(deg_ref, x_ref, w1_ref, g1_ref, g1b_ref, dinv_ref):
    deg = deg_ref[0] + deg_ref[1]
    dinv = lax.rsqrt(deg + 1.0)
    dinv_ref[...] = dinv
    h = jnp.dot(x_ref[...], w1_ref[...], preferred_element_type=jnp.float32)
    g1 = h * dinv[:N]
    g1_ref[...] = g1
    lo = lax.convert_element_type(
        lax.bitcast_convert_type(g1[:, :D_HID // 2].astype(jnp.bfloat16),
                                 jnp.uint16), jnp.int32)
    hi = lax.convert_element_type(
        lax.bitcast_convert_type(g1[:, D_HID // 2:].astype(jnp.bfloat16),
                                 jnp.uint16), jnp.int32)
    g1b_ref[...] = lo | (hi << 16)


def _tc_mid_body(s1_ref, g1_ref, dinv_ref, b1_ref, w2_ref, g2_ref):
    s1 = s1_ref[0, :N] + s1_ref[1, :N]
    dinv = dinv_ref[:N]
    out1 = jnp.maximum(dinv * (s1 + g1_ref[...]) + b1_ref[...], 0.0)
    h2 = jnp.dot(out1, w2_ref[...], preferred_element_type=jnp.float32)
    g2_ref[...] = h2 * dinv


def _tc_out_body(s2_ref, g2_ref, dinv_ref, b2_ref, o_ref):
    s2 = s2_ref[0, :N] + s2_ref[1, :N]
    z = dinv_ref[:N] * (s2 + g2_ref[...]) + b2_ref[...]
    m = jnp.max(z, axis=1, keepdims=True)
    lse = m + jnp.log(jnp.sum(jnp.exp(z - m), axis=1, keepdims=True))
    o_ref[...] = z - lse


def kernel(x, G2_edge_attr, G1_edge_attr_matrix, G3_edge_index, G3_edge_attr,
           W1, b1, W2, b2):
    src = G3_edge_index[0]
    dst = G3_edge_index[1]
    ew = G3_edge_attr
    pad = EPAD - E
    zi = jnp.zeros((pad,), jnp.int32)
    src2d = jnp.concatenate([src, zi]).reshape(EPAD // 128, 128)
    dst2d = jnp.concatenate([dst, zi]).reshape(EPAD // 128, 128)
    ewp = jnp.concatenate([ew, jnp.zeros((pad,), jnp.float32)])

    deg_part = _sc_deg(dst2d, ewp, jnp.zeros((NACC,), jnp.float32))
    deg_col = deg_part.reshape(NC, NACC, 1)

    g1, g1b, dinv = pl.pallas_call(
        _tc_pre_body,
        out_shape=(jax.ShapeDtypeStruct((N, D_HID), jnp.float32),
                   jax.ShapeDtypeStruct((N, D_HID // 2), jnp.int32),
                   jax.ShapeDtypeStruct((NACC, 1), jnp.float32)),
    )(deg_col, x, W1)

    s1 = _sc_agg(D_HID, True, 256, src2d, dst2d, ewp, g1b,
                 jnp.zeros((NACC, D_HID), jnp.float32))

    g2 = pl.pallas_call(
        _tc_mid_body,
        out_shape=jax.ShapeDtypeStruct((N, D_OUT), jnp.float32),
    )(s1, g1, dinv, b1.reshape(1, D_HID), W2)

    s2 = _sc_agg(D_OUT, False, 512, src2d, dst2d, ewp, g2,
                 jnp.zeros((NACC, D_OUT), jnp.float32))

    out = pl.pallas_call(
        _tc_out_body,
        out_shape=jax.ShapeDtypeStruct((N, D_OUT), jnp.float32),
    )(s2, g2, dinv, b2.reshape(1, D_OUT))
    return out


# speedup vs baseline: 28.5061x; 1.1217x over previous
import functools

import jax
import jax.numpy as jnp
from jax import lax
from jax.experimental import pallas as pl
from jax.experimental.pallas import tpu as pltpu
from jax.experimental.pallas import tpu_sc as plsc

N = 10000
E = 320000
D_IN = 128
D_HID = 64
D_OUT = 16

NC = 2
NS = 16
NW = NC * NS
CHUNK = 512
EPT = 10240
EPAD = NW * EPT
NCHUNK = EPT // CHUNK
NACC = 10240
RPT = NACC // NS

_mesh = plsc.VectorSubcoreMesh(core_axis_name="c", subcore_axis_name="s")


def _sc_deg_body(ei2, ew, zeros, out, dstv, ewv, sem_i, sem_s, deg_sh):
    c = lax.axis_index("c")
    s = lax.axis_index("s")
    w = c * NS + s
    nj = CHUNK // 128
    rb_max = (E - CHUNK) // 128
    pltpu.sync_copy(zeros.at[pl.ds(s * RPT, RPT)], deg_sh.at[pl.ds(s * RPT, RPT)])
    plsc.subcore_barrier()

    def start_idx(k, b):
        rb = jnp.minimum(w * (EPT // 128) + k * nj, rb_max)
        base = w * EPT + k * CHUNK
        pltpu.async_copy(ei2.at[1, pl.ds(rb, nj), :], dstv.at[b], sem_i)
        pltpu.async_copy(ew.at[pl.ds(base, CHUNK)],
                         ewv.at[pl.ds(b * CHUNK, CHUNK)], sem_i)

    def wait_idx(b):
        pltpu.make_async_copy(ei2.at[1, pl.ds(0, nj), :], dstv.at[b],
                              sem_i).wait()
        pltpu.make_async_copy(ew.at[pl.ds(0, CHUNK)],
                              ewv.at[pl.ds(b * CHUNK, CHUNK)], sem_i).wait()

    def start_scatter(b):
        for j in range(nj):
            pltpu.async_copy(ewv.at[pl.ds(b * CHUNK + j * 128, 128)],
                             deg_sh.at[dstv.at[b, j]], sem_s, add=True)

    def wait_scatter(b):
        for j in range(nj):
            pltpu.make_async_copy(ewv.at[pl.ds(b * CHUNK + j * 128, 128)],
                                  deg_sh.at[dstv.at[b, j]], sem_s).wait()

    start_idx(0, 0)
    for k in range(NCHUNK):
        if k >= 2:
            wait_scatter((k - 2) % 3)
        if k + 1 < NCHUNK:
            start_idx(k + 1, (k + 1) % 3)
        wait_idx(k % 3)
        start_scatter(k % 3)
    wait_scatter((NCHUNK - 2) % 3)
    wait_scatter((NCHUNK - 1) % 3)
    plsc.subcore_barrier()
    pltpu.sync_copy(deg_sh.at[pl.ds(s * RPT, RPT)], out.at[c, pl.ds(s * RPT, RPT)])


def _sc_agg_body(d, packed, chunk, *refs):
    if packed:
        (ei2, ew, g, zeros, out, srcv, dstv, ewv, rows, rows_b,
         sem_i, sem_g, sem_s, acc_sh) = refs
    else:
        (ei2, ew, g, zeros, out, srcv, dstv, ewv, rows,
         sem_i, sem_g, sem_s, acc_sh) = refs
        rows_b = rows
    c = lax.axis_index("c")
    s = lax.axis_index("s")
    w = c * NS + s
    pltpu.sync_copy(zeros.at[pl.ds(s * RPT, RPT), :],
                    acc_sh.at[pl.ds(s * RPT, RPT), :])
    plsc.subcore_barrier()
    nchunk = EPT // chunk
    nj = chunk // 128
    G = 4
    outer = nchunk // G

    rb_max = (E - chunk) // 128

    def start_idx(k, b):
        rb = jnp.minimum(w * (EPT // 128) + k * nj, rb_max)
        base = w * EPT + k * chunk
        pltpu.async_copy(ei2.at[0, pl.ds(rb, nj), :], srcv.at[b], sem_i)
        pltpu.async_copy(ei2.at[1, pl.ds(rb, nj), :], dstv.at[b], sem_i)
        pltpu.async_copy(ew.at[pl.ds(base, chunk)],
                         ewv.at[b, pl.ds(0, chunk)], sem_i)

    def wait_idx(b):
        pltpu.make_async_copy(ei2.at[0, pl.ds(0, nj), :], srcv.at[b],
                              sem_i).wait()
        pltpu.make_async_copy(ei2.at[1, pl.ds(0, nj), :], dstv.at[b],
                              sem_i).wait()
        pltpu.make_async_copy(ew.at[pl.ds(0, chunk)],
                              ewv.at[b, pl.ds(0, chunk)], sem_i).wait()

    def start_gather(b, rb):
        for j in range(nj):
            pltpu.async_copy(g.at[srcv.at[b, j]],
                             rows_b.at[rb, pl.ds(j * 128, 128), :], sem_g)

    def wait_gather(b, rb):
        for j in range(nj):
            pltpu.make_async_copy(g.at[srcv.at[b, j]],
                                  rows_b.at[rb, pl.ds(j * 128, 128), :],
                                  sem_g).wait()

    def start_scatter(b, rb):
        for j in range(nj):
            pltpu.async_copy(rows.at[rb, pl.ds(j * 128, 128), :],
                             acc_sh.at[dstv.at[b, j]], sem_s, add=True)

    def wait_scatter(b, rb):
        for j in range(nj):
            pltpu.make_async_copy(rows.at[rb, pl.ds(j * 128, 128), :],
                                  acc_sh.at[dstv.at[b, j]], sem_s).wait()

    def scale_chunk(b, rb):
        def scale(i, _):
            ws = ewv[b, pl.ds(i, 16)][0]
            if packed:
                for jj in range(d // 32):
                    wv = rows_b[rb, i, pl.ds(jj * 16, 16)]
                    e0 = lax.bitcast_convert_type(wv << 16, jnp.float32)
                    e1 = lax.bitcast_convert_type(wv & jnp.int32(-65536),
                                                  jnp.float32)
                    rows[rb, i, pl.ds(jj * 16, 16)] = e0 * ws
                    rows[rb, i, pl.ds(d // 2 + jj * 16, 16)] = e1 * ws
            else:
                for jj in range(d // 16):
                    rows[rb, i, pl.ds(jj * 16, 16)] = (
                        rows[rb, i, pl.ds(jj * 16, 16)] * ws)
            return 0
        lax.fori_loop(0, chunk, scale, 0, unroll=8)

    start_idx(0, 0)
    wait_idx(0)
    start_gather(0, 0)

    def outer_body(kk, _):
        for gg in range(G):
            k = kk * G + gg
            start_idx(k + 1, (gg + 1) % G)
            wait_gather(gg, gg % 2)
            wait_idx((gg + 1) % G)

            @pl.when(k >= 1)
            def _():
                wait_scatter((gg + 1) % G, (gg + 1) % 2)
            start_gather((gg + 1) % G, (gg + 1) % 2)
            scale_chunk(gg, gg % 2)
            start_scatter(gg, gg % 2)
        return 0

    lax.fori_loop(0, outer - 1, outer_body, 0)

    for gg in range(G):
        k = (outer - 1) * G + gg
        if gg + 1 < G:
            start_idx(k + 1, gg + 1)
        wait_gather(gg, gg % 2)
        if gg + 1 < G:
            wait_idx(gg + 1)
            wait_scatter(gg + 1, (gg + 1) % 2)
            start_gather(gg + 1, (gg + 1) % 2)
        scale_chunk(gg, gg % 2)
        if gg + 1 == G:
            wait_scatter((gg + 1) % G, (gg + 1) % 2)
        start_scatter(gg, gg % 2)
    wait_scatter(G - 1, (G - 1) % 2)
    plsc.subcore_barrier()
    pltpu.sync_copy(acc_sh.at[pl.ds(s * RPT, RPT), :],
                    out.at[c, pl.ds(s * RPT, RPT), :])


def _sc_deg(ei2, ew, zeros):
    return pl.kernel(
        _sc_deg_body,
        out_type=jax.ShapeDtypeStruct((NC, NACC), jnp.float32),
        mesh=_mesh,
        scratch_types=[
            pltpu.VMEM((3, CHUNK // 128, 128), jnp.int32),
            pltpu.VMEM((3 * CHUNK,), jnp.float32),
            pltpu.SemaphoreType.DMA,
            pltpu.SemaphoreType.DMA,
            pltpu.VMEM_SHARED((NACC,), jnp.float32),
        ],
        name="gcn_sc_deg",
    )(ei2, ew, zeros)


def _sc_agg(d, packed, chunk, ei2, ew, g, zeros):
    scratch = [
        pltpu.VMEM((4, chunk // 128, 128), jnp.int32),
        pltpu.VMEM((4, chunk // 128, 128), jnp.int32),
        pltpu.VMEM((4, chunk + 16), jnp.float32),
        pltpu.VMEM((2, chunk, d), jnp.float32),
    ]
    if packed:
        scratch.append(pltpu.VMEM((2, chunk, d // 2), jnp.int32))
    scratch += [
        pltpu.SemaphoreType.DMA,
        pltpu.SemaphoreType.DMA,
        pltpu.SemaphoreType.DMA,
        pltpu.VMEM_SHARED((NACC, d), jnp.float32),
    ]
    return pl.kernel(
        functools.partial(_sc_agg_body, d, packed, chunk),
        out_type=jax.ShapeDtypeStruct((NC, NACC, d), jnp.float32),
        mesh=_mesh,
        scratch_types=scratch,
        compiler_params=pltpu.CompilerParams(use_tc_tiling_on_sc=False),
        name=f"gcn_sc_agg{d}",
    )(ei2, ew, g, zeros)


def _tc_pre_body(deg_ref, x_ref, w1_ref, g1_ref, g1b_ref, dinv_ref):
    deg = deg_ref[0] + deg_ref[1]
    dinv = lax.rsqrt(deg + 1.0)
    dinv_ref[...] = dinv
    h = jnp.dot(x_ref[...], w1_ref[...], preferred_element_type=jnp.float32)
    g1 = h * dinv[:N]
    g1_ref[...] = g1
    lo = lax.convert_element_type(
        lax.bitcast_convert_type(g1[:, :D_HID // 2].astype(jnp.bfloat16),
                                 jnp.uint16), jnp.int32)
    hi = lax.convert_element_type(
        lax.bitcast_convert_type(g1[:, D_HID // 2:].astype(jnp.bfloat16),
                                 jnp.uint16), jnp.int32)
    g1b_ref[...] = lo | (hi << 16)


def _tc_mid_body(s1_ref, g1_ref, dinv_ref, b1_ref, w2_ref, g2_ref):
    s1 = s1_ref[0, :N] + s1_ref[1, :N]
    dinv = dinv_ref[:N]
    out1 = jnp.maximum(dinv * (s1 + g1_ref[...]) + b1_ref[...], 0.0)
    h2 = jnp.dot(out1, w2_ref[...], preferred_element_type=jnp.float32)
    g2_ref[...] = h2 * dinv


def _tc_out_body(s2_ref, g2_ref, dinv_ref, b2_ref, o_ref):
    s2 = s2_ref[0, :N] + s2_ref[1, :N]
    z = dinv_ref[:N] * (s2 + g2_ref[...]) + b2_ref[...]
    m = jnp.max(z, axis=1, keepdims=True)
    lse = m + jnp.log(jnp.sum(jnp.exp(z - m), axis=1, keepdims=True))
    o_ref[...] = z - lse


def kernel(x, G2_edge_attr, G1_edge_attr_matrix, G3_edge_index, G3_edge_attr,
           W1, b1, W2, b2):
    src = G3_edge_index[0]
    dst = G3_edge_index[1]
    ew = G3_edge_attr
    ei2 = G3_edge_index.reshape(2, E // 128, 128)
    ewp = jnp.concatenate([ew, jnp.zeros((EPAD - E,), jnp.float32)])

    deg_part = _sc_deg(ei2, ewp, jnp.zeros((NACC,), jnp.float32))
    deg_col = deg_part.reshape(NC, NACC, 1)

    g1, g1b, dinv = pl.pallas_call(
        _tc_pre_body,
        out_shape=(jax.ShapeDtypeStruct((N, D_HID), jnp.float32),
                   jax.ShapeDtypeStruct((N, D_HID // 2), jnp.int32),
                   jax.ShapeDtypeStruct((NACC, 1), jnp.float32)),
    )(deg_col, x, W1)

    s1 = _sc_agg(D_HID, True, 256, ei2, ewp, g1b,
                 jnp.zeros((NACC, D_HID), jnp.float32))

    g2 = pl.pallas_call(
        _tc_mid_body,
        out_shape=jax.ShapeDtypeStruct((N, D_OUT), jnp.float32),
    )(s1, g1, dinv, b1.reshape(1, D_HID), W2)

    s2 = _sc_agg(D_OUT, False, 512, ei2, ewp, g2,
                 jnp.zeros((NACC, D_OUT), jnp.float32))

    out = pl.pallas_call(
        _tc_out_body,
        out_shape=jax.ShapeDtypeStruct((N, D_OUT), jnp.float32),
    )(s2, g2, dinv, b2.reshape(1, D_OUT))
    return out
